# baseline (speedup 1.0000x reference)
import functools

import jax
import jax.numpy as jnp
from jax import lax
from jax.experimental import pallas as pl
from jax.experimental.pallas import tpu as pltpu
from jax.experimental.pallas import tpu_sc as plsc

NUM_EMB = 100000
SEQ = 12
IMG = 32
D = IMG * IMG
BATCH = 1024

_NC, _NS = 2, 16
_NW = _NC * _NS
_B_PER_W = BATCH // _NW


@functools.lru_cache(maxsize=None)
def _make_sc_gather():
    @functools.partial(
        pl.kernel,
        mesh=plsc.VectorSubcoreMesh(core_axis_name="c", subcore_axis_name="s"),
        out_type=jax.ShapeDtypeStruct((BATCH, D), jnp.float32),
        scratch_types=[
            pltpu.VMEM((_B_PER_W,), jnp.int32),
            pltpu.VMEM((_B_PER_W, D), jnp.float32),
            pltpu.SemaphoreType.DMA,
        ],
    )
    def _sc_gather(table_hbm, idx_hbm, out_hbm, idx_v, rows_v, sem):
        wid = lax.axis_index("s") * _NC + lax.axis_index("c")
        base = wid * _B_PER_W
        pltpu.sync_copy(idx_hbm.at[pl.ds(base, _B_PER_W)], idx_v)
        pltpu.async_copy(table_hbm.at[idx_v], rows_v, sem).wait()
        pltpu.sync_copy(rows_v, out_hbm.at[pl.ds(base, _B_PER_W)])

    return _sc_gather


_BB = 8


def _tc_assemble_body(x_ref, emb_ref, out_ref):
    out_ref[:, :3] = x_ref[...]
    emb = emb_ref[...]
    out_ref[:, 3] = jnp.broadcast_to(emb[:, None, :], (_BB, SEQ, D))


def _tc_assemble(xf, emb):
    return pl.pallas_call(
        _tc_assemble_body,
        grid=(BATCH // _BB,),
        in_specs=[
            pl.BlockSpec((_BB, 3, SEQ, D), lambda i: (i, 0, 0, 0)),
            pl.BlockSpec((_BB, D), lambda i: (i, 0)),
        ],
        out_specs=pl.BlockSpec((_BB, 4, SEQ, D), lambda i: (i, 0, 0, 0)),
        out_shape=jax.ShapeDtypeStruct((BATCH, 4, SEQ, D), jnp.float32),
        compiler_params=pltpu.CompilerParams(
            dimension_semantics=("arbitrary",),
        ),
    )(xf, emb)


def kernel(x, id, table):
    xf = x.reshape(BATCH, 3, SEQ, D)
    emb = _make_sc_gather()(table, id)
    out = _tc_assemble(xf, emb)
    return out.reshape(BATCH, 4, SEQ, IMG, IMG)


# speedup vs baseline: 1.3407x; 1.3407x over previous
import functools

import jax
import jax.numpy as jnp
from jax import lax
from jax.experimental import pallas as pl
from jax.experimental.pallas import tpu as pltpu
from jax.experimental.pallas import tpu_sc as plsc

NUM_EMB = 100000
SEQ = 12
IMG = 32
D = IMG * IMG
BATCH = 1024

_NC, _NS = 2, 16
_NW = _NC * _NS
_B_PER_W = BATCH // _NW


@functools.lru_cache(maxsize=None)
def _make_sc_gather():
    @functools.partial(
        pl.kernel,
        mesh=plsc.VectorSubcoreMesh(core_axis_name="c", subcore_axis_name="s"),
        out_type=jax.ShapeDtypeStruct((BATCH, D), jnp.float32),
        scratch_types=[
            pltpu.VMEM((_B_PER_W,), jnp.int32),
            pltpu.VMEM((_B_PER_W, D), jnp.float32),
            pltpu.SemaphoreType.DMA,
        ],
    )
    def _sc_gather(table_hbm, idx_hbm, out_hbm, idx_v, rows_v, sem):
        wid = lax.axis_index("s") * _NC + lax.axis_index("c")
        base = wid * _B_PER_W
        pltpu.sync_copy(idx_hbm.at[pl.ds(base, _B_PER_W)], idx_v)
        pltpu.async_copy(table_hbm.at[idx_v], rows_v, sem).wait()
        pltpu.sync_copy(rows_v, out_hbm.at[pl.ds(base, _B_PER_W)])

    return _sc_gather


_BB = 16
_XROW = 3 * SEQ * D
_OROW = 4 * SEQ * D


def _tc_assemble_body(x_ref, emb_ref, out_ref):
    out_ref[:, :_XROW] = x_ref[...]
    emb = emb_ref[...]
    for s in range(SEQ):
        out_ref[:, _XROW + s * D : _XROW + (s + 1) * D] = emb


def _tc_assemble(xf, emb):
    return pl.pallas_call(
        _tc_assemble_body,
        grid=(BATCH // _BB,),
        in_specs=[
            pl.BlockSpec((_BB, _XROW), lambda i: (i, 0)),
            pl.BlockSpec((_BB, D), lambda i: (i, 0)),
        ],
        out_specs=pl.BlockSpec((_BB, _OROW), lambda i: (i, 0)),
        out_shape=jax.ShapeDtypeStruct((BATCH, _OROW), jnp.float32),
        compiler_params=pltpu.CompilerParams(
            dimension_semantics=("parallel",),
        ),
    )(xf, emb)


def kernel(x, id, table):
    xf = x.reshape(BATCH, _XROW)
    emb = _make_sc_gather()(table, id)
    out = _tc_assemble(xf, emb)
    return out.reshape(BATCH, 4, SEQ, IMG, IMG)
